# Initial kernel scaffold; baseline (speedup 1.0000x reference)
import jax
import jax.numpy as jnp
from jax.experimental import pallas as pl


def kernel(x, edge_index, batch, W_in, b_in, conv_W, conv_b, bn_gamma, bn_beta, W1, b1, W2, b2, W3, b3):
    raise NotImplementedError("write your pallas kernel here")


# speedup vs baseline: 8.1496x; 8.1496x over previous
import dataclasses
import functools

import jax
import jax.numpy as jnp
from jax import lax
from jax.experimental import pallas as pl
from jax.experimental.pallas import tpu as pltpu
from jax.experimental.pallas import tpu_sc as plsc

N = 10000
E = 320000
H = 128
G = 16
L = 4
EPS = 1e-5

NC = 2
NS = 16
NW = NC * NS
HH = H // NC
CHUNK = 128
NCH = 80
E_PAD = NW * NCH * CHUNK
NCH16 = 160
E_PAD16 = NS * NCH16 * CHUNK
NP = 10112
ROWS_PER_SUB = NP // NS


def _sc_degree_body(dst_hbm, out_hbm, dst_v, hist_v):
    c = lax.axis_index("c")
    s = lax.axis_index("s")
    w = c * NS + s

    @pl.loop(0, NP // 16)
    def _(i):
        hist_v[pl.ds(i * 16, 16)] = jnp.zeros((16,), jnp.float32)

    pltpu.sync_copy(dst_hbm.at[w], dst_v)
    ones16 = jnp.ones((16,), jnp.float32)

    @pl.loop(0, NCH)
    def _(j):
        for k in range(0, CHUNK, 16):
            idx = dst_v[j, pl.ds(k, 16)]
            plsc.addupdate_scatter(hist_v, [idx], ones16)

    pltpu.sync_copy(hist_v, out_hbm.at[w])


def _sc_scatter_body(mp_hbm, src_hbm, dst_hbm, out_hbm, src_v, dst_v, rows_v,
                     zero_v, acc, sem):
    c = lax.axis_index("c")
    s = lax.axis_index("s")

    @pl.loop(0, CHUNK)
    def _(i):
        for j in range(0, HH, 16):
            zero_v[i, pl.ds(j, 16)] = jnp.zeros((16,), jnp.float32)

    base = s * ROWS_PER_SUB
    off = 0
    for sz in (CHUNK, CHUNK, CHUNK, CHUNK, ROWS_PER_SUB - 4 * CHUNK):
        pltpu.sync_copy(zero_v.at[pl.ds(0, sz)], acc.at[pl.ds(base + off, sz)])
        off += sz

    pltpu.sync_copy(src_hbm.at[s], src_v)
    pltpu.sync_copy(dst_hbm.at[s], dst_v)
    plsc.subcore_barrier()

    mp_c = mp_hbm.at[c]

    @pl.loop(0, NCH16)
    def _(j):
        pltpu.async_copy(mp_c.at[src_v.at[j]], rows_v, sem).wait()
        pltpu.sync_copy(rows_v, acc.at[dst_v.at[j]], add=True)

    plsc.subcore_barrier()
    pltpu.sync_copy(
        acc.at[pl.ds(base, ROWS_PER_SUB)],
        out_hbm.at[c, pl.ds(base, ROWS_PER_SUB)],
    )


def _sc_params(**kw):
    cp = pltpu.CompilerParams(**kw)
    if "needs_layout_passes" in pltpu.CompilerParams.__dataclass_fields__:
        cp = dataclasses.replace(cp, needs_layout_passes=False)
    return cp


@functools.cache
def _sc_kernels():
    mesh = plsc.VectorSubcoreMesh(
        core_axis_name="c", subcore_axis_name="s", num_cores=NC,
        num_subcores=NS,
    )
    sc_degree = pl.kernel(
        _sc_degree_body,
        out_type=jax.ShapeDtypeStruct((NW, NP), jnp.float32),
        mesh=mesh,
        compiler_params=_sc_params(),
        scratch_types=[
            pltpu.VMEM((NCH, CHUNK), jnp.int32),
            pltpu.VMEM((NP,), jnp.float32),
        ],
    )
    sc_scatter = pl.kernel(
        _sc_scatter_body,
        out_type=jax.ShapeDtypeStruct((NC, NP, HH), jnp.float32),
        mesh=mesh,
        compiler_params=_sc_params(use_tc_tiling_on_sc=False),
        scratch_types=[
            pltpu.VMEM((NCH16, CHUNK), jnp.int32),
            pltpu.VMEM((NCH16, CHUNK), jnp.int32),
            pltpu.VMEM((CHUNK, HH), jnp.float32),
            pltpu.VMEM((CHUNK, HH), jnp.float32),
            pltpu.VMEM_SHARED((NP, HH), jnp.float32),
            pltpu.SemaphoreType.DMA,
        ],
    )
    return sc_degree, sc_scatter


def _store_mp(mp_ref, mp):
    zpad = jnp.zeros((NP - N, HH), jnp.float32)
    for c in range(NC):
        mp_ref[c, 0:N, :] = mp[:, c * HH:(c + 1) * HH]
        mp_ref[c, N:NP, :] = zpad


def _agg_from(parts_ref, mp_ref, dis):
    halves = [parts_ref[c, 0:N, :] + mp_ref[c, 0:N, :] for c in range(NC)]
    return dis * jnp.concatenate(halves, axis=1)


def _tc_pre_body(x_ref, win_ref, bin_ref, w0_ref, degp_ref,
                 h_ref, mp_ref, dis_ref):
    deg = jnp.sum(degp_ref[...], axis=0)[0:N, None] + 1.0
    dis = lax.rsqrt(deg)
    dis_ref[...] = dis
    h = jax.nn.relu(
        jnp.dot(x_ref[...], win_ref[...], preferred_element_type=jnp.float32)
        + bin_ref[...]
    )
    h_ref[...] = h
    mp = jnp.dot(h, w0_ref[...], preferred_element_type=jnp.float32) * dis
    _store_mp(mp_ref, mp)


def _tc_mid_body(h_ref, mp_ref, parts_ref, dis_ref, wn_ref, gs_ref,
                 cb_ref, hn_ref, mpn_ref):
    dis = dis_ref[...]
    agg = _agg_from(parts_ref, mp_ref, dis)
    y = jax.nn.relu(gs_ref[...] * agg + cb_ref[...])
    hn = h_ref[...] + y
    hn_ref[...] = hn
    mpn = jnp.dot(hn, wn_ref[...], preferred_element_type=jnp.float32) * dis
    _store_mp(mpn_ref, mpn)


def _tc_post_body(h_ref, mp_ref, parts_ref, dis_ref, gs_ref, cb_ref,
                  batch_ref, w1_ref, b1_ref, w2_ref, b2_ref, w3_ref, b3_ref,
                  out_ref):
    dis = dis_ref[...]
    agg = _agg_from(parts_ref, mp_ref, dis)
    y = jax.nn.relu(gs_ref[...] * agg + cb_ref[...])
    h = h_ref[...] + y
    gid = lax.broadcasted_iota(jnp.int32, (G, N), 0)
    onehot = jnp.where(batch_ref[...] == gid, 1.0, 0.0).astype(jnp.float32)
    sums = jnp.dot(onehot, h, preferred_element_type=jnp.float32)
    counts = jnp.sum(onehot, axis=1, keepdims=True)
    g = sums / jnp.maximum(counts, 1.0)
    z = jax.nn.relu(
        jnp.dot(g, w1_ref[...], preferred_element_type=jnp.float32)
        + b1_ref[...]
    )
    z = jax.nn.relu(
        jnp.dot(z, w2_ref[...], preferred_element_type=jnp.float32)
        + b2_ref[...]
    )
    out_ref[...] = (
        jnp.dot(z, w3_ref[...], preferred_element_type=jnp.float32)
        + b3_ref[...]
    )


_tc_pre = pl.pallas_call(
    _tc_pre_body,
    out_shape=(
        jax.ShapeDtypeStruct((N, H), jnp.float32),
        jax.ShapeDtypeStruct((NC, NP, HH), jnp.float32),
        jax.ShapeDtypeStruct((N, 1), jnp.float32),
    ),
)

_tc_mid = pl.pallas_call(
    _tc_mid_body,
    out_shape=(
        jax.ShapeDtypeStruct((N, H), jnp.float32),
        jax.ShapeDtypeStruct((NC, NP, HH), jnp.float32),
    ),
)

_tc_post = pl.pallas_call(
    _tc_post_body,
    out_shape=jax.ShapeDtypeStruct((G, 1), jnp.float32),
)


def kernel(x, edge_index, batch, W_in, b_in, conv_W, conv_b, bn_gamma,
           bn_beta, W1, b1, W2, b2, W3, b3):
    pad = E_PAD - E
    dst32 = jnp.concatenate([edge_index[1], jnp.full((pad,), N, jnp.int32)])
    dst32 = dst32.reshape(NW, NCH, CHUNK)
    pad16 = E_PAD16 - E
    src = jnp.concatenate([edge_index[0], jnp.full((pad16,), N, jnp.int32)])
    dst = jnp.concatenate([edge_index[1], jnp.full((pad16,), N, jnp.int32)])
    src = src.reshape(NS, NCH16, CHUNK)
    dst = dst.reshape(NS, NCH16, CHUNK)

    inv = 1.0 / jnp.sqrt(1.0 + EPS)
    gs = bn_gamma * inv
    cb = gs * conv_b + bn_beta

    sc_degree, sc_scatter = _sc_kernels()
    degp = sc_degree(dst32)
    h, mp, dis = _tc_pre(x, W_in, b_in.reshape(1, H), conv_W[0], degp)
    for l in range(L):
        parts = sc_scatter(mp, src, dst)
        if l + 1 < L:
            h, mp = _tc_mid(h, mp, parts, dis, conv_W[l + 1],
                            gs[l].reshape(1, H), cb[l].reshape(1, H))
        else:
            out = _tc_post(h, mp, parts, dis,
                           gs[l].reshape(1, H), cb[l].reshape(1, H),
                           batch.reshape(1, N), W1, b1.reshape(1, -1),
                           W2, b2.reshape(1, -1), W3, b3.reshape(1, 1))
    return out


# speedup vs baseline: 8.1586x; 1.0011x over previous
import dataclasses
import functools

import jax
import jax.numpy as jnp
from jax import lax
from jax.experimental import pallas as pl
from jax.experimental.pallas import tpu as pltpu
from jax.experimental.pallas import tpu_sc as plsc

N = 10000
E = 320000
H = 128
G = 16
L = 4
EPS = 1e-5

NC = 2
NS = 16
NW = NC * NS
HH = H // NC
CHUNK = 128
NCH = 80
E_PAD = NW * NCH * CHUNK
NCH16 = 160
E_PAD16 = NS * NCH16 * CHUNK
NP = 10112
ROWS_PER_SUB = NP // NS


def _sc_degree_body(dst_hbm, out_hbm, dst_v, ones_v, zero_v, acc):
    c = lax.axis_index("c")
    s = lax.axis_index("s")
    w = c * NS + s

    @pl.loop(0, CHUNK)
    def _(i):
        ones_v[i, pl.ds(0, 16)] = jnp.ones((16,), jnp.float32)

    @pl.loop(0, ROWS_PER_SUB)
    def _(i):
        zero_v[i, pl.ds(0, 16)] = jnp.zeros((16,), jnp.float32)

    pltpu.sync_copy(zero_v, acc.at[pl.ds(s * ROWS_PER_SUB, ROWS_PER_SUB)])
    pltpu.sync_copy(dst_hbm.at[w], dst_v)
    plsc.subcore_barrier()

    @pl.loop(0, NCH)
    def _(j):
        pltpu.sync_copy(ones_v, acc.at[dst_v.at[j]], add=True)

    plsc.subcore_barrier()
    pltpu.sync_copy(
        acc.at[pl.ds(s * ROWS_PER_SUB, ROWS_PER_SUB)],
        out_hbm.at[c, pl.ds(s * ROWS_PER_SUB, ROWS_PER_SUB)],
    )


def _sc_scatter_body(mp_hbm, src_hbm, dst_hbm, out_hbm, src_v, dst_v, rows_v,
                     zero_v, acc, sem):
    c = lax.axis_index("c")
    s = lax.axis_index("s")

    @pl.loop(0, CHUNK)
    def _(i):
        for j in range(0, HH, 16):
            zero_v[i, pl.ds(j, 16)] = jnp.zeros((16,), jnp.float32)

    base = s * ROWS_PER_SUB
    off = 0
    for sz in (CHUNK, CHUNK, CHUNK, CHUNK, ROWS_PER_SUB - 4 * CHUNK):
        pltpu.sync_copy(zero_v.at[pl.ds(0, sz)], acc.at[pl.ds(base + off, sz)])
        off += sz

    pltpu.sync_copy(src_hbm.at[s], src_v)
    pltpu.sync_copy(dst_hbm.at[s], dst_v)
    plsc.subcore_barrier()

    mp_c = mp_hbm.at[c]

    @pl.loop(0, NCH16)
    def _(j):
        pltpu.async_copy(mp_c.at[src_v.at[j]], rows_v, sem).wait()
        pltpu.sync_copy(rows_v, acc.at[dst_v.at[j]], add=True)

    plsc.subcore_barrier()
    pltpu.sync_copy(
        acc.at[pl.ds(base, ROWS_PER_SUB)],
        out_hbm.at[c, pl.ds(base, ROWS_PER_SUB)],
    )


def _sc_params(**kw):
    cp = pltpu.CompilerParams(**kw)
    if "needs_layout_passes" in pltpu.CompilerParams.__dataclass_fields__:
        cp = dataclasses.replace(cp, needs_layout_passes=False)
    return cp


@functools.cache
def _sc_kernels():
    mesh = plsc.VectorSubcoreMesh(
        core_axis_name="c", subcore_axis_name="s", num_cores=NC,
        num_subcores=NS,
    )
    sc_degree = pl.kernel(
        _sc_degree_body,
        out_type=jax.ShapeDtypeStruct((NC, NP, 16), jnp.float32),
        mesh=mesh,
        compiler_params=_sc_params(use_tc_tiling_on_sc=False),
        scratch_types=[
            pltpu.VMEM((NCH, CHUNK), jnp.int32),
            pltpu.VMEM((CHUNK, 16), jnp.float32),
            pltpu.VMEM((ROWS_PER_SUB, 16), jnp.float32),
            pltpu.VMEM_SHARED((NP, 16), jnp.float32),
        ],
    )
    sc_scatter = pl.kernel(
        _sc_scatter_body,
        out_type=jax.ShapeDtypeStruct((NC, NP, HH), jnp.float32),
        mesh=mesh,
        compiler_params=_sc_params(use_tc_tiling_on_sc=False),
        scratch_types=[
            pltpu.VMEM((NCH16, CHUNK), jnp.int32),
            pltpu.VMEM((NCH16, CHUNK), jnp.int32),
            pltpu.VMEM((CHUNK, HH), jnp.float32),
            pltpu.VMEM((CHUNK, HH), jnp.float32),
            pltpu.VMEM_SHARED((NP, HH), jnp.float32),
            pltpu.SemaphoreType.DMA,
        ],
    )
    return sc_degree, sc_scatter


def _store_mp(mp_ref, mp):
    zpad = jnp.zeros((NP - N, HH), jnp.float32)
    for c in range(NC):
        mp_ref[c, 0:N, :] = mp[:, c * HH:(c + 1) * HH]
        mp_ref[c, N:NP, :] = zpad


def _agg_from(parts_ref, mp_ref, dis):
    halves = [parts_ref[c, 0:N, :] + mp_ref[c, 0:N, :] for c in range(NC)]
    return dis * jnp.concatenate(halves, axis=1)


def _tc_pre_body(x_ref, win_ref, bin_ref, w0_ref, degp_ref,
                 h_ref, mp_ref, dis_ref):
    deg = degp_ref[0, 0:N, 0:1] + degp_ref[1, 0:N, 0:1] + 1.0
    dis = lax.rsqrt(deg)
    dis_ref[...] = dis
    h = jax.nn.relu(
        jnp.dot(x_ref[...], win_ref[...], preferred_element_type=jnp.float32)
        + bin_ref[...]
    )
    h_ref[...] = h
    mp = jnp.dot(h, w0_ref[...], preferred_element_type=jnp.float32) * dis
    _store_mp(mp_ref, mp)


def _tc_mid_body(h_ref, mp_ref, parts_ref, dis_ref, wn_ref, gs_ref,
                 cb_ref, hn_ref, mpn_ref):
    dis = dis_ref[...]
    agg = _agg_from(parts_ref, mp_ref, dis)
    y = jax.nn.relu(gs_ref[...] * agg + cb_ref[...])
    hn = h_ref[...] + y
    hn_ref[...] = hn
    mpn = jnp.dot(hn, wn_ref[...], preferred_element_type=jnp.float32) * dis
    _store_mp(mpn_ref, mpn)


def _tc_post_body(h_ref, mp_ref, parts_ref, dis_ref, gs_ref, cb_ref,
                  batch_ref, w1_ref, b1_ref, w2_ref, b2_ref, w3_ref, b3_ref,
                  out_ref):
    dis = dis_ref[...]
    agg = _agg_from(parts_ref, mp_ref, dis)
    y = jax.nn.relu(gs_ref[...] * agg + cb_ref[...])
    h = h_ref[...] + y
    gid = lax.broadcasted_iota(jnp.int32, (G, N), 0)
    onehot = jnp.where(batch_ref[...] == gid, 1.0, 0.0).astype(jnp.float32)
    sums = jnp.dot(onehot, h, preferred_element_type=jnp.float32)
    counts = jnp.sum(onehot, axis=1, keepdims=True)
    g = sums / jnp.maximum(counts, 1.0)
    z = jax.nn.relu(
        jnp.dot(g, w1_ref[...], preferred_element_type=jnp.float32)
        + b1_ref[...]
    )
    z = jax.nn.relu(
        jnp.dot(z, w2_ref[...], preferred_element_type=jnp.float32)
        + b2_ref[...]
    )
    out_ref[...] = (
        jnp.dot(z, w3_ref[...], preferred_element_type=jnp.float32)
        + b3_ref[...]
    )


_tc_pre = pl.pallas_call(
    _tc_pre_body,
    out_shape=(
        jax.ShapeDtypeStruct((N, H), jnp.float32),
        jax.ShapeDtypeStruct((NC, NP, HH), jnp.float32),
        jax.ShapeDtypeStruct((N, 1), jnp.float32),
    ),
)

_tc_mid = pl.pallas_call(
    _tc_mid_body,
    out_shape=(
        jax.ShapeDtypeStruct((N, H), jnp.float32),
        jax.ShapeDtypeStruct((NC, NP, HH), jnp.float32),
    ),
)

_tc_post = pl.pallas_call(
    _tc_post_body,
    out_shape=jax.ShapeDtypeStruct((G, 1), jnp.float32),
)


def kernel(x, edge_index, batch, W_in, b_in, conv_W, conv_b, bn_gamma,
           bn_beta, W1, b1, W2, b2, W3, b3):
    pad = E_PAD - E
    dst32 = jnp.concatenate([edge_index[1], jnp.full((pad,), N, jnp.int32)])
    dst32 = dst32.reshape(NW, NCH, CHUNK)
    pad16 = E_PAD16 - E
    src = jnp.concatenate([edge_index[0], jnp.full((pad16,), N, jnp.int32)])
    dst = jnp.concatenate([edge_index[1], jnp.full((pad16,), N, jnp.int32)])
    src = src.reshape(NS, NCH16, CHUNK)
    dst = dst.reshape(NS, NCH16, CHUNK)

    inv = 1.0 / jnp.sqrt(1.0 + EPS)
    gs = bn_gamma * inv
    cb = gs * conv_b + bn_beta

    sc_degree, sc_scatter = _sc_kernels()
    degp = sc_degree(dst32)
    h, mp, dis = _tc_pre(x, W_in, b_in.reshape(1, H), conv_W[0], degp)
    for l in range(L):
        parts = sc_scatter(mp, src, dst)
        if l + 1 < L:
            h, mp = _tc_mid(h, mp, parts, dis, conv_W[l + 1],
                            gs[l].reshape(1, H), cb[l].reshape(1, H))
        else:
            out = _tc_post(h, mp, parts, dis,
                           gs[l].reshape(1, H), cb[l].reshape(1, H),
                           batch.reshape(1, N), W1, b1.reshape(1, -1),
                           W2, b2.reshape(1, -1), W3, b3.reshape(1, 1))
    return out


# speedup vs baseline: 10.1990x; 1.2501x over previous
import dataclasses
import functools

import jax
import jax.numpy as jnp
from jax import lax
from jax.experimental import pallas as pl
from jax.experimental.pallas import tpu as pltpu
from jax.experimental.pallas import tpu_sc as plsc

N = 10000
E = 320000
H = 128
G = 16
L = 4
EPS = 1e-5

NC = 2
NS = 16
NW = NC * NS
HH = H // NC
CHUNK = 128
NCH = 80
E_PAD = NW * NCH * CHUNK
NCH16 = 160
E_PAD16 = NS * NCH16 * CHUNK
NP = 10112
ROWS_PER_SUB = NP // NS


def _sc_degree_body(dst_hbm, out_hbm, dst_v, ones_v, zero_v, acc):
    c = lax.axis_index("c")
    s = lax.axis_index("s")
    w = c * NS + s

    @pl.loop(0, CHUNK)
    def _(i):
        ones_v[i, pl.ds(0, 16)] = jnp.ones((16,), jnp.float32)

    @pl.loop(0, ROWS_PER_SUB)
    def _(i):
        zero_v[i, pl.ds(0, 16)] = jnp.zeros((16,), jnp.float32)

    pltpu.sync_copy(zero_v, acc.at[pl.ds(s * ROWS_PER_SUB, ROWS_PER_SUB)])
    pltpu.sync_copy(dst_hbm.at[w], dst_v)
    plsc.subcore_barrier()

    @pl.loop(0, NCH)
    def _(j):
        pltpu.sync_copy(ones_v, acc.at[dst_v.at[j]], add=True)

    plsc.subcore_barrier()
    pltpu.sync_copy(
        acc.at[pl.ds(s * ROWS_PER_SUB, ROWS_PER_SUB)],
        out_hbm.at[c, pl.ds(s * ROWS_PER_SUB, ROWS_PER_SUB)],
    )


def _sc_scatter_body(mp_hbm, src_hbm, dst_hbm, out_hbm, src_v, dst_v, rows_v,
                     zero_v, acc, sem):
    c = lax.axis_index("c")
    s = lax.axis_index("s")

    @pl.loop(0, CHUNK)
    def _(i):
        for j in range(0, HH, 16):
            zero_v[i, pl.ds(j, 16)] = jnp.zeros((16,), jnp.float32)

    base = s * ROWS_PER_SUB
    off = 0
    for sz in (CHUNK, CHUNK, CHUNK, CHUNK, ROWS_PER_SUB - 4 * CHUNK):
        pltpu.sync_copy(zero_v.at[pl.ds(0, sz)], acc.at[pl.ds(base + off, sz)])
        off += sz

    pltpu.sync_copy(src_hbm.at[s], src_v)
    pltpu.sync_copy(dst_hbm.at[s], dst_v)
    plsc.subcore_barrier()

    mp_c = mp_hbm.at[c]
    rows0 = rows_v.at[0]
    rows1 = rows_v.at[1]
    sem0 = sem.at[0]
    sem1 = sem.at[1]

    gather0 = pltpu.async_copy(mp_c.at[src_v.at[0]], rows0, sem0)
    gather0.wait()
    pltpu.async_copy(mp_c.at[src_v.at[1]], rows1, sem1)

    @pl.loop(0, NCH16, step=2)
    def _(j):
        pltpu.sync_copy(rows0, acc.at[dst_v.at[j]], add=True)
        pltpu.make_async_copy(mp_c.at[src_v.at[j]], rows1, sem1).wait()

        @pl.when(j + 2 < NCH16)
        def _():
            pltpu.async_copy(mp_c.at[src_v.at[j + 2]], rows0, sem0)

        pltpu.sync_copy(rows1, acc.at[dst_v.at[j + 1]], add=True)

        @pl.when(j + 3 < NCH16)
        def _():
            pltpu.async_copy(mp_c.at[src_v.at[j + 3]], rows1, sem1)

        @pl.when(j + 2 < NCH16)
        def _():
            pltpu.make_async_copy(mp_c.at[src_v.at[j]], rows0, sem0).wait()

    plsc.subcore_barrier()
    pltpu.sync_copy(
        acc.at[pl.ds(base, ROWS_PER_SUB)],
        out_hbm.at[c, pl.ds(base, ROWS_PER_SUB)],
    )


def _sc_params(**kw):
    cp = pltpu.CompilerParams(**kw)
    if "needs_layout_passes" in pltpu.CompilerParams.__dataclass_fields__:
        cp = dataclasses.replace(cp, needs_layout_passes=False)
    return cp


@functools.cache
def _sc_kernels():
    mesh = plsc.VectorSubcoreMesh(
        core_axis_name="c", subcore_axis_name="s", num_cores=NC,
        num_subcores=NS,
    )
    sc_degree = pl.kernel(
        _sc_degree_body,
        out_type=jax.ShapeDtypeStruct((NC, NP, 16), jnp.float32),
        mesh=mesh,
        compiler_params=_sc_params(use_tc_tiling_on_sc=False),
        scratch_types=[
            pltpu.VMEM((NCH, CHUNK), jnp.int32),
            pltpu.VMEM((CHUNK, 16), jnp.float32),
            pltpu.VMEM((ROWS_PER_SUB, 16), jnp.float32),
            pltpu.VMEM_SHARED((NP, 16), jnp.float32),
        ],
    )
    sc_scatter = pl.kernel(
        _sc_scatter_body,
        out_type=jax.ShapeDtypeStruct((NC, NP, HH), jnp.float32),
        mesh=mesh,
        compiler_params=_sc_params(use_tc_tiling_on_sc=False),
        scratch_types=[
            pltpu.VMEM((NCH16, CHUNK), jnp.int32),
            pltpu.VMEM((NCH16, CHUNK), jnp.int32),
            pltpu.VMEM((2, CHUNK, HH), jnp.float32),
            pltpu.VMEM((CHUNK, HH), jnp.float32),
            pltpu.VMEM_SHARED((NP, HH), jnp.float32),
            pltpu.SemaphoreType.DMA((2,)),
        ],
    )
    return sc_degree, sc_scatter


def _store_mp(mp_ref, mp):
    zpad = jnp.zeros((NP - N, HH), jnp.float32)
    for c in range(NC):
        mp_ref[c, 0:N, :] = mp[:, c * HH:(c + 1) * HH]
        mp_ref[c, N:NP, :] = zpad


def _agg_from(parts_ref, mp_ref, dis):
    halves = [parts_ref[c, 0:N, :] + mp_ref[c, 0:N, :] for c in range(NC)]
    return dis * jnp.concatenate(halves, axis=1)


def _tc_pre_body(x_ref, win_ref, bin_ref, w0_ref, degp_ref,
                 h_ref, mp_ref, dis_ref):
    deg = degp_ref[0, 0:N, 0:1] + degp_ref[1, 0:N, 0:1] + 1.0
    dis = lax.rsqrt(deg)
    dis_ref[...] = dis
    h = jax.nn.relu(
        jnp.dot(x_ref[...], win_ref[...], preferred_element_type=jnp.float32)
        + bin_ref[...]
    )
    h_ref[...] = h
    mp = jnp.dot(h, w0_ref[...], preferred_element_type=jnp.float32) * dis
    _store_mp(mp_ref, mp)


def _tc_mid_body(h_ref, mp_ref, parts_ref, dis_ref, wn_ref, gs_ref,
                 cb_ref, hn_ref, mpn_ref):
    dis = dis_ref[...]
    agg = _agg_from(parts_ref, mp_ref, dis)
    y = jax.nn.relu(gs_ref[...] * agg + cb_ref[...])
    hn = h_ref[...] + y
    hn_ref[...] = hn
    mpn = jnp.dot(hn, wn_ref[...], preferred_element_type=jnp.float32) * dis
    _store_mp(mpn_ref, mpn)


def _tc_post_body(h_ref, mp_ref, parts_ref, dis_ref, gs_ref, cb_ref,
                  batch_ref, w1_ref, b1_ref, w2_ref, b2_ref, w3_ref, b3_ref,
                  out_ref):
    dis = dis_ref[...]
    agg = _agg_from(parts_ref, mp_ref, dis)
    y = jax.nn.relu(gs_ref[...] * agg + cb_ref[...])
    h = h_ref[...] + y
    gid = lax.broadcasted_iota(jnp.int32, (G, N), 0)
    onehot = jnp.where(batch_ref[...] == gid, 1.0, 0.0).astype(jnp.float32)
    sums = jnp.dot(onehot, h, preferred_element_type=jnp.float32)
    counts = jnp.sum(onehot, axis=1, keepdims=True)
    g = sums / jnp.maximum(counts, 1.0)
    z = jax.nn.relu(
        jnp.dot(g, w1_ref[...], preferred_element_type=jnp.float32)
        + b1_ref[...]
    )
    z = jax.nn.relu(
        jnp.dot(z, w2_ref[...], preferred_element_type=jnp.float32)
        + b2_ref[...]
    )
    out_ref[...] = (
        jnp.dot(z, w3_ref[...], preferred_element_type=jnp.float32)
        + b3_ref[...]
    )


_tc_pre = pl.pallas_call(
    _tc_pre_body,
    out_shape=(
        jax.ShapeDtypeStruct((N, H), jnp.float32),
        jax.ShapeDtypeStruct((NC, NP, HH), jnp.float32),
        jax.ShapeDtypeStruct((N, 1), jnp.float32),
    ),
)

_tc_mid = pl.pallas_call(
    _tc_mid_body,
    out_shape=(
        jax.ShapeDtypeStruct((N, H), jnp.float32),
        jax.ShapeDtypeStruct((NC, NP, HH), jnp.float32),
    ),
)

_tc_post = pl.pallas_call(
    _tc_post_body,
    out_shape=jax.ShapeDtypeStruct((G, 1), jnp.float32),
)


def kernel(x, edge_index, batch, W_in, b_in, conv_W, conv_b, bn_gamma,
           bn_beta, W1, b1, W2, b2, W3, b3):
    pad = E_PAD - E
    dst32 = jnp.concatenate([edge_index[1], jnp.full((pad,), N, jnp.int32)])
    dst32 = dst32.reshape(NW, NCH, CHUNK)
    pad16 = E_PAD16 - E
    src = jnp.concatenate([edge_index[0], jnp.full((pad16,), N, jnp.int32)])
    dst = jnp.concatenate([edge_index[1], jnp.full((pad16,), N, jnp.int32)])
    src = src.reshape(NS, NCH16, CHUNK)
    dst = dst.reshape(NS, NCH16, CHUNK)

    inv = 1.0 / jnp.sqrt(1.0 + EPS)
    gs = bn_gamma * inv
    cb = gs * conv_b + bn_beta

    sc_degree, sc_scatter = _sc_kernels()
    degp = sc_degree(dst32)
    h, mp, dis = _tc_pre(x, W_in, b_in.reshape(1, H), conv_W[0], degp)
    for l in range(L):
        parts = sc_scatter(mp, src, dst)
        if l + 1 < L:
            h, mp = _tc_mid(h, mp, parts, dis, conv_W[l + 1],
                            gs[l].reshape(1, H), cb[l].reshape(1, H))
        else:
            out = _tc_post(h, mp, parts, dis,
                           gs[l].reshape(1, H), cb[l].reshape(1, H),
                           batch.reshape(1, N), W1, b1.reshape(1, -1),
                           W2, b2.reshape(1, -1), W3, b3.reshape(1, 1))
    return out


# speedup vs baseline: 20.3522x; 1.9955x over previous
import dataclasses
import functools

import jax
import jax.numpy as jnp
from jax import lax
from jax.experimental import pallas as pl
from jax.experimental.pallas import tpu as pltpu
from jax.experimental.pallas import tpu_sc as plsc

N = 10000
E = 320000
H = 128
G = 16
L = 4
EPS = 1e-5

NC = 2
NS = 16
NW = NC * NS
HH = H // NC
CHUNK = 128
NCH = 80
E_PAD = NW * NCH * CHUNK
NCH16 = 160
E_PAD16 = NS * NCH16 * CHUNK
NP = 10112
ROWS_PER_SUB = NP // NS


def _sc_degree_body(dst_hbm, out_hbm, dst_v, ones_v, zero_v, acc):
    c = lax.axis_index("c")
    s = lax.axis_index("s")
    w = c * NS + s

    @pl.loop(0, CHUNK)
    def _(i):
        ones_v[i, pl.ds(0, 16)] = jnp.ones((16,), jnp.float32)

    @pl.loop(0, ROWS_PER_SUB)
    def _(i):
        zero_v[i, pl.ds(0, 16)] = jnp.zeros((16,), jnp.float32)

    pltpu.sync_copy(zero_v, acc.at[pl.ds(s * ROWS_PER_SUB, ROWS_PER_SUB)])
    pltpu.sync_copy(dst_hbm.at[w], dst_v)
    plsc.subcore_barrier()

    @pl.loop(0, NCH)
    def _(j):
        pltpu.sync_copy(ones_v, acc.at[dst_v.at[j]], add=True)

    plsc.subcore_barrier()
    pltpu.sync_copy(
        acc.at[pl.ds(s * ROWS_PER_SUB, ROWS_PER_SUB)],
        out_hbm.at[c, pl.ds(s * ROWS_PER_SUB, ROWS_PER_SUB)],
    )


def _sc_scatter_body(mp_hbm, src_hbm, dst_hbm, out_hbm, src_v, dst_v, rows_v,
                     zero_v, acc, sem):
    c = lax.axis_index("c")
    s = lax.axis_index("s")

    @pl.loop(0, CHUNK)
    def _(i):
        for j in range(0, HH, 16):
            zero_v[i, pl.ds(j, 16)] = jnp.zeros((16,), jnp.float32)

    base = s * ROWS_PER_SUB
    off = 0
    for sz in (CHUNK, CHUNK, CHUNK, CHUNK, ROWS_PER_SUB - 4 * CHUNK):
        pltpu.sync_copy(zero_v.at[pl.ds(0, sz)], acc.at[pl.ds(base + off, sz)])
        off += sz

    pltpu.sync_copy(src_hbm.at[s], src_v)
    pltpu.sync_copy(dst_hbm.at[s], dst_v)
    plsc.subcore_barrier()

    mp_c = mp_hbm.at[c]
    rows0 = rows_v.at[0]
    rows1 = rows_v.at[1]
    sem0 = sem.at[0]
    sem1 = sem.at[1]

    gather0 = pltpu.async_copy(mp_c.at[src_v.at[0]], rows0, sem0)
    gather0.wait()
    pltpu.async_copy(mp_c.at[src_v.at[1]], rows1, sem1)

    @pl.loop(0, NCH16, step=2)
    def _(j):
        pltpu.sync_copy(rows0, acc.at[dst_v.at[j]], add=True)
        pltpu.make_async_copy(mp_c.at[src_v.at[j]], rows1, sem1).wait()

        @pl.when(j + 2 < NCH16)
        def _():
            pltpu.async_copy(mp_c.at[src_v.at[j + 2]], rows0, sem0)

        pltpu.sync_copy(rows1, acc.at[dst_v.at[j + 1]], add=True)

        @pl.when(j + 3 < NCH16)
        def _():
            pltpu.async_copy(mp_c.at[src_v.at[j + 3]], rows1, sem1)

        @pl.when(j + 2 < NCH16)
        def _():
            pltpu.make_async_copy(mp_c.at[src_v.at[j]], rows0, sem0).wait()

    plsc.subcore_barrier()
    pltpu.sync_copy(
        acc.at[pl.ds(base, ROWS_PER_SUB)],
        out_hbm.at[c, pl.ds(base, ROWS_PER_SUB)],
    )


def _sc_params(**kw):
    cp = pltpu.CompilerParams(**kw)
    if "needs_layout_passes" in pltpu.CompilerParams.__dataclass_fields__:
        cp = dataclasses.replace(cp, needs_layout_passes=False)
    return cp


@functools.cache
def _sc_kernels():
    mesh = plsc.VectorSubcoreMesh(
        core_axis_name="c", subcore_axis_name="s", num_cores=NC,
        num_subcores=NS,
    )
    sc_degree = pl.kernel(
        _sc_degree_body,
        out_type=jax.ShapeDtypeStruct((NC, NP, 16), jnp.float32),
        mesh=mesh,
        compiler_params=_sc_params(use_tc_tiling_on_sc=False),
        scratch_types=[
            pltpu.VMEM((NCH, CHUNK), jnp.int32),
            pltpu.VMEM((CHUNK, 16), jnp.float32),
            pltpu.VMEM((ROWS_PER_SUB, 16), jnp.float32),
            pltpu.VMEM_SHARED((NP, 16), jnp.float32),
        ],
    )
    sc_scatter = pl.kernel(
        _sc_scatter_body,
        out_type=jax.ShapeDtypeStruct((NC, NP, HH), jnp.float32),
        mesh=mesh,
        compiler_params=_sc_params(use_tc_tiling_on_sc=False),
        scratch_types=[
            pltpu.VMEM((NCH16, CHUNK), jnp.int32),
            pltpu.VMEM((NCH16, CHUNK), jnp.int32),
            pltpu.VMEM((2, CHUNK, HH), jnp.float32),
            pltpu.VMEM((CHUNK, HH), jnp.float32),
            pltpu.VMEM_SHARED((NP, HH), jnp.float32),
            pltpu.SemaphoreType.DMA((2,)),
        ],
    )
    return sc_degree, sc_scatter


def _store_mp(mp_ref, mp):
    zpad = jnp.zeros((NP - N, HH), jnp.float32)
    for c in range(NC):
        mp_ref[c, 0:N, :] = mp[:, c * HH:(c + 1) * HH]
        mp_ref[c, N:NP, :] = zpad


def _agg_from(parts_ref, mp_ref, dis):
    halves = [parts_ref[c, 0:N, :] + mp_ref[c, 0:N, :] for c in range(NC)]
    return dis * jnp.concatenate(halves, axis=1)


def _tc_pre_body(x_ref, win_ref, bin_ref, w0_ref, degp_ref,
                 h_ref, mp_ref, dis_ref):
    deg = degp_ref[0, 0:N, 0:1] + degp_ref[1, 0:N, 0:1] + 1.0
    dis = lax.rsqrt(deg)
    dis_ref[...] = dis
    h = jax.nn.relu(
        jnp.dot(x_ref[...], win_ref[...], preferred_element_type=jnp.float32)
        + bin_ref[...]
    )
    h_ref[...] = h
    mp = jnp.dot(h, w0_ref[...], preferred_element_type=jnp.float32) * dis
    _store_mp(mp_ref, mp)


def _tc_mid_body(h_ref, mp_ref, parts_ref, dis_ref, wn_ref, gs_ref,
                 cb_ref, hn_ref, mpn_ref):
    dis = dis_ref[...]
    agg = _agg_from(parts_ref, mp_ref, dis)
    y = jax.nn.relu(gs_ref[...] * agg + cb_ref[...])
    hn = h_ref[...] + y
    hn_ref[...] = hn
    mpn = jnp.dot(hn, wn_ref[...], preferred_element_type=jnp.float32) * dis
    _store_mp(mpn_ref, mpn)


def _tc_post_body(h_ref, mp_ref, parts_ref, dis_ref, gs_ref, cb_ref,
                  batch_ref, w1_ref, b1_ref, w2_ref, b2_ref, w3_ref, b3_ref,
                  out_ref):
    dis = dis_ref[...]
    agg = _agg_from(parts_ref, mp_ref, dis)
    y = jax.nn.relu(gs_ref[...] * agg + cb_ref[...])
    h = h_ref[...] + y
    gid = lax.broadcasted_iota(jnp.int32, (G, N), 0)
    onehot = jnp.where(batch_ref[...] == gid, 1.0, 0.0).astype(jnp.float32)
    sums = jnp.dot(onehot, h, preferred_element_type=jnp.float32)
    counts = jnp.sum(onehot, axis=1, keepdims=True)
    g = sums / jnp.maximum(counts, 1.0)
    z = jax.nn.relu(
        jnp.dot(g, w1_ref[...], preferred_element_type=jnp.float32)
        + b1_ref[...]
    )
    z = jax.nn.relu(
        jnp.dot(z, w2_ref[...], preferred_element_type=jnp.float32)
        + b2_ref[...]
    )
    out_ref[...] = (
        jnp.dot(z, w3_ref[...], preferred_element_type=jnp.float32)
        + b3_ref[...]
    )


_tc_pre = pl.pallas_call(
    _tc_pre_body,
    out_shape=(
        jax.ShapeDtypeStruct((N, H), jnp.float32),
        jax.ShapeDtypeStruct((NC, NP, HH), jnp.float32),
        jax.ShapeDtypeStruct((N, 1), jnp.float32),
    ),
)

_tc_mid = pl.pallas_call(
    _tc_mid_body,
    out_shape=(
        jax.ShapeDtypeStruct((N, H), jnp.float32),
        jax.ShapeDtypeStruct((NC, NP, HH), jnp.float32),
    ),
)

_tc_post = pl.pallas_call(
    _tc_post_body,
    out_shape=jax.ShapeDtypeStruct((G, 1), jnp.float32),
)


def kernel(x, edge_index, batch, W_in, b_in, conv_W, conv_b, bn_gamma,
           bn_beta, W1, b1, W2, b2, W3, b3):
    pad = E_PAD - E
    padv = N + (jnp.arange(pad, dtype=jnp.int32) % (NP - N))
    dst32 = jnp.concatenate([edge_index[1], padv])
    dst32 = dst32.reshape(NW, NCH, CHUNK)
    src = jnp.concatenate([edge_index[0], padv])
    dst = jnp.concatenate([edge_index[1], padv])
    src = src.reshape(NS, NCH16, CHUNK)
    dst = dst.reshape(NS, NCH16, CHUNK)

    inv = 1.0 / jnp.sqrt(1.0 + EPS)
    gs = bn_gamma * inv
    cb = gs * conv_b + bn_beta

    sc_degree, sc_scatter = _sc_kernels()
    degp = sc_degree(dst32)
    h, mp, dis = _tc_pre(x, W_in, b_in.reshape(1, H), conv_W[0], degp)
    for l in range(L):
        parts = sc_scatter(mp, src, dst)
        if l + 1 < L:
            h, mp = _tc_mid(h, mp, parts, dis, conv_W[l + 1],
                            gs[l].reshape(1, H), cb[l].reshape(1, H))
        else:
            out = _tc_post(h, mp, parts, dis,
                           gs[l].reshape(1, H), cb[l].reshape(1, H),
                           batch.reshape(1, N), W1, b1.reshape(1, -1),
                           W2, b2.reshape(1, -1), W3, b3.reshape(1, 1))
    return out


# speedup vs baseline: 22.3589x; 1.0986x over previous
import dataclasses
import functools

import jax
import jax.numpy as jnp
from jax import lax
from jax.experimental import pallas as pl
from jax.experimental.pallas import tpu as pltpu
from jax.experimental.pallas import tpu_sc as plsc

N = 10000
E = 320000
H = 128
G = 16
L = 4
EPS = 1e-5

NC = 2
NS = 16
NW = NC * NS
HH = H // NC
CHUNK = 128
NBUF = 4
NCH = 80
E_PAD = NW * NCH * CHUNK
NCH16 = 160
E_PAD16 = NS * NCH16 * CHUNK
NP = 10112
ROWS_PER_SUB = NP // NS


def _sc_degree_body(dst_hbm, out_hbm, dst_v, ones_v, zero_v, acc):
    c = lax.axis_index("c")
    s = lax.axis_index("s")
    w = c * NS + s

    @pl.loop(0, CHUNK)
    def _(i):
        ones_v[i, pl.ds(0, 16)] = jnp.ones((16,), jnp.float32)

    @pl.loop(0, ROWS_PER_SUB)
    def _(i):
        zero_v[i, pl.ds(0, 16)] = jnp.zeros((16,), jnp.float32)

    pltpu.sync_copy(zero_v, acc.at[pl.ds(s * ROWS_PER_SUB, ROWS_PER_SUB)])
    pltpu.sync_copy(dst_hbm.at[w], dst_v)
    plsc.subcore_barrier()

    @pl.loop(0, NCH)
    def _(j):
        pltpu.sync_copy(ones_v, acc.at[dst_v.at[j]], add=True)

    plsc.subcore_barrier()
    pltpu.sync_copy(
        acc.at[pl.ds(s * ROWS_PER_SUB, ROWS_PER_SUB)],
        out_hbm.at[c, pl.ds(s * ROWS_PER_SUB, ROWS_PER_SUB)],
    )


def _sc_scatter_body(mp_hbm, src_hbm, dst_hbm, out_hbm, src_v, dst_v, rows_v,
                     zero_v, acc, gsems, ssems):
    c = lax.axis_index("c")
    s = lax.axis_index("s")

    @pl.loop(0, CHUNK)
    def _(i):
        for j in range(0, HH, 16):
            zero_v[i, pl.ds(j, 16)] = jnp.zeros((16,), jnp.float32)

    base = s * ROWS_PER_SUB
    off = 0
    for sz in (CHUNK, CHUNK, CHUNK, CHUNK, ROWS_PER_SUB - 4 * CHUNK):
        pltpu.sync_copy(zero_v.at[pl.ds(0, sz)], acc.at[pl.ds(base + off, sz)])
        off += sz

    pltpu.sync_copy(src_hbm.at[s], src_v)
    pltpu.sync_copy(dst_hbm.at[s], dst_v)
    plsc.subcore_barrier()

    mp_c = mp_hbm.at[c]
    rows = [rows_v.at[k] for k in range(NBUF)]
    gsem = [gsems.at[k] for k in range(NBUF)]
    ssem = [ssems.at[k] for k in range(NBUF)]

    for k in range(NBUF):
        pltpu.async_copy(mp_c.at[src_v.at[k]], rows[k], gsem[k])

    @pl.loop(0, NCH16, step=NBUF)
    def _(j):
        for k in range(NBUF):
            pltpu.make_async_copy(mp_c.at[src_v.at[0]], rows[k],
                                  gsem[k]).wait()
            pltpu.async_copy(rows[k], acc.at[dst_v.at[j + k]], ssem[k],
                             add=True)
        for k in range(NBUF):
            @pl.when(j + NBUF + k < NCH16)
            def _(k=k):
                pltpu.make_async_copy(rows[k], acc.at[dst_v.at[0]],
                                      ssem[k]).wait()
                pltpu.async_copy(mp_c.at[src_v.at[j + NBUF + k]], rows[k],
                                 gsem[k])

    for k in range(NBUF):
        pltpu.make_async_copy(rows[k], acc.at[dst_v.at[0]], ssem[k]).wait()

    plsc.subcore_barrier()
    pltpu.sync_copy(
        acc.at[pl.ds(base, ROWS_PER_SUB)],
        out_hbm.at[c, pl.ds(base, ROWS_PER_SUB)],
    )


def _sc_params(**kw):
    cp = pltpu.CompilerParams(**kw)
    if "needs_layout_passes" in pltpu.CompilerParams.__dataclass_fields__:
        cp = dataclasses.replace(cp, needs_layout_passes=False)
    return cp


@functools.cache
def _sc_kernels():
    mesh = plsc.VectorSubcoreMesh(
        core_axis_name="c", subcore_axis_name="s", num_cores=NC,
        num_subcores=NS,
    )
    sc_degree = pl.kernel(
        _sc_degree_body,
        out_type=jax.ShapeDtypeStruct((NC, NP, 16), jnp.float32),
        mesh=mesh,
        compiler_params=_sc_params(use_tc_tiling_on_sc=False),
        scratch_types=[
            pltpu.VMEM((NCH, CHUNK), jnp.int32),
            pltpu.VMEM((CHUNK, 16), jnp.float32),
            pltpu.VMEM((ROWS_PER_SUB, 16), jnp.float32),
            pltpu.VMEM_SHARED((NP, 16), jnp.float32),
        ],
    )
    sc_scatter = pl.kernel(
        _sc_scatter_body,
        out_type=jax.ShapeDtypeStruct((NC, NP, HH), jnp.float32),
        mesh=mesh,
        compiler_params=_sc_params(use_tc_tiling_on_sc=False),
        scratch_types=[
            pltpu.VMEM((NCH16, CHUNK), jnp.int32),
            pltpu.VMEM((NCH16, CHUNK), jnp.int32),
            pltpu.VMEM((NBUF, CHUNK, HH), jnp.float32),
            pltpu.VMEM((CHUNK, HH), jnp.float32),
            pltpu.VMEM_SHARED((NP, HH), jnp.float32),
            pltpu.SemaphoreType.DMA((NBUF,)),
            pltpu.SemaphoreType.DMA((NBUF,)),
        ],
    )
    return sc_degree, sc_scatter


def _store_mp(mp_ref, mp):
    zpad = jnp.zeros((NP - N, HH), jnp.float32)
    for c in range(NC):
        mp_ref[c, 0:N, :] = mp[:, c * HH:(c + 1) * HH]
        mp_ref[c, N:NP, :] = zpad


def _agg_from(parts_ref, mp_ref, dis):
    halves = [parts_ref[c, 0:N, :] + mp_ref[c, 0:N, :] for c in range(NC)]
    return dis * jnp.concatenate(halves, axis=1)


def _tc_pre_body(x_ref, win_ref, bin_ref, w0_ref, degp_ref,
                 h_ref, mp_ref, dis_ref):
    deg = degp_ref[0, 0:N, 0:1] + degp_ref[1, 0:N, 0:1] + 1.0
    dis = lax.rsqrt(deg)
    dis_ref[...] = dis
    h = jax.nn.relu(
        jnp.dot(x_ref[...], win_ref[...], preferred_element_type=jnp.float32)
        + bin_ref[...]
    )
    h_ref[...] = h
    mp = jnp.dot(h, w0_ref[...], preferred_element_type=jnp.float32) * dis
    _store_mp(mp_ref, mp)


def _tc_mid_body(h_ref, mp_ref, parts_ref, dis_ref, wn_ref, gs_ref,
                 cb_ref, hn_ref, mpn_ref):
    dis = dis_ref[...]
    agg = _agg_from(parts_ref, mp_ref, dis)
    y = jax.nn.relu(gs_ref[...] * agg + cb_ref[...])
    hn = h_ref[...] + y
    hn_ref[...] = hn
    mpn = jnp.dot(hn, wn_ref[...], preferred_element_type=jnp.float32) * dis
    _store_mp(mpn_ref, mpn)


def _tc_post_body(h_ref, mp_ref, parts_ref, dis_ref, gs_ref, cb_ref,
                  batch_ref, w1_ref, b1_ref, w2_ref, b2_ref, w3_ref, b3_ref,
                  out_ref):
    dis = dis_ref[...]
    agg = _agg_from(parts_ref, mp_ref, dis)
    y = jax.nn.relu(gs_ref[...] * agg + cb_ref[...])
    h = h_ref[...] + y
    gid = lax.broadcasted_iota(jnp.int32, (G, N), 0)
    onehot = jnp.where(batch_ref[...] == gid, 1.0, 0.0).astype(jnp.float32)
    sums = jnp.dot(onehot, h, preferred_element_type=jnp.float32)
    counts = jnp.sum(onehot, axis=1, keepdims=True)
    g = sums / jnp.maximum(counts, 1.0)
    z = jax.nn.relu(
        jnp.dot(g, w1_ref[...], preferred_element_type=jnp.float32)
        + b1_ref[...]
    )
    z = jax.nn.relu(
        jnp.dot(z, w2_ref[...], preferred_element_type=jnp.float32)
        + b2_ref[...]
    )
    out_ref[...] = (
        jnp.dot(z, w3_ref[...], preferred_element_type=jnp.float32)
        + b3_ref[...]
    )


_tc_pre = pl.pallas_call(
    _tc_pre_body,
    out_shape=(
        jax.ShapeDtypeStruct((N, H), jnp.float32),
        jax.ShapeDtypeStruct((NC, NP, HH), jnp.float32),
        jax.ShapeDtypeStruct((N, 1), jnp.float32),
    ),
)

_tc_mid = pl.pallas_call(
    _tc_mid_body,
    out_shape=(
        jax.ShapeDtypeStruct((N, H), jnp.float32),
        jax.ShapeDtypeStruct((NC, NP, HH), jnp.float32),
    ),
)

_tc_post = pl.pallas_call(
    _tc_post_body,
    out_shape=jax.ShapeDtypeStruct((G, 1), jnp.float32),
)


def kernel(x, edge_index, batch, W_in, b_in, conv_W, conv_b, bn_gamma,
           bn_beta, W1, b1, W2, b2, W3, b3):
    pad = E_PAD - E
    padv = N + (jnp.arange(pad, dtype=jnp.int32) % (NP - N))
    dst32 = jnp.concatenate([edge_index[1], padv])
    dst32 = dst32.reshape(NW, NCH, CHUNK)
    src = jnp.concatenate([edge_index[0], padv])
    dst = jnp.concatenate([edge_index[1], padv])
    src = src.reshape(NS, NCH16, CHUNK)
    dst = dst.reshape(NS, NCH16, CHUNK)

    inv = 1.0 / jnp.sqrt(1.0 + EPS)
    gs = bn_gamma * inv
    cb = gs * conv_b + bn_beta

    sc_degree, sc_scatter = _sc_kernels()
    degp = sc_degree(dst32)
    h, mp, dis = _tc_pre(x, W_in, b_in.reshape(1, H), conv_W[0], degp)
    for l in range(L):
        parts = sc_scatter(mp, src, dst)
        if l + 1 < L:
            h, mp = _tc_mid(h, mp, parts, dis, conv_W[l + 1],
                            gs[l].reshape(1, H), cb[l].reshape(1, H))
        else:
            out = _tc_post(h, mp, parts, dis,
                           gs[l].reshape(1, H), cb[l].reshape(1, H),
                           batch.reshape(1, N), W1, b1.reshape(1, -1),
                           W2, b2.reshape(1, -1), W3, b3.reshape(1, 1))
    return out
